# Initial kernel scaffold; baseline (speedup 1.0000x reference)
import jax
import jax.numpy as jnp
from jax.experimental import pallas as pl


def kernel(x, idx, conv_w, conv_b):
    raise NotImplementedError("write your pallas kernel here")


# speedup vs baseline: 1.5030x; 1.5030x over previous
import functools

import jax
import jax.numpy as jnp
from jax import lax
from jax.experimental import pallas as pl
from jax.experimental.pallas import tpu as pltpu
from jax.experimental.pallas import tpu_sc as plsc

NC = 2
NS = 16
NW = NC * NS
LANES = 16

C = 128
K = 32
CH = 4
CG = C // LANES


def _sc_gather_max(x_flat, idx3, n_pad):
    npw = n_pad // NW
    nch = npw // CH
    mesh = plsc.VectorSubcoreMesh(core_axis_name="c", subcore_axis_name="s")

    @functools.partial(
        pl.kernel,
        out_type=jax.ShapeDtypeStruct((n_pad, C), jnp.float32),
        mesh=mesh,
        scratch_types=[
            pltpu.VMEM((nch, CH * K), jnp.int32),
            pltpu.VMEM((2, CH * K, C), jnp.float32),
            pltpu.VMEM((CH, C), jnp.float32),
            pltpu.SemaphoreType.DMA,
            pltpu.SemaphoreType.DMA,
        ],
    )
    def k(x_hbm, idx_hbm, agg_hbm, idx_v, rows_v, out_v, sem0, sem1):
        wid = lax.axis_index("s") * NC + lax.axis_index("c")
        pltpu.sync_copy(idx_hbm.at[wid], idx_v)

        @pl.loop(0, nch)
        def _chunk(c):
            pltpu.async_copy(x_hbm.at[idx_v.at[c]], rows_v.at[0], sem0).wait()
            for j in range(CH):
                for cg in range(CG):
                    acc = rows_v[0, j * K, pl.ds(cg * LANES, LANES)]
                    for kk in range(1, K):
                        acc = jnp.maximum(
                            acc, rows_v[0, j * K + kk, pl.ds(cg * LANES, LANES)])
                    out_v[j, pl.ds(cg * LANES, LANES)] = acc
            pltpu.sync_copy(out_v, agg_hbm.at[pl.ds(wid * npw + c * CH, CH)])

    return k(x_flat, idx3)


def _tc_conv(agg, w, b2, n):
    def body(agg_ref, w_ref, b_ref, out_ref):
        prod = lax.dot_general(
            w_ref[...], agg_ref[...], (((1,), (1,)), ((), ())),
            preferred_element_type=jnp.float32)
        out_ref[...] = jnp.maximum(prod + b_ref[...], 0.0)

    return pl.pallas_call(
        body,
        out_shape=jax.ShapeDtypeStruct((C, n), jnp.float32),
    )(agg[:n], w, b2)


def kernel(x, idx, conv_w, conv_b):
    B_, C_, N_ = x.shape
    n_pad = ((N_ + (NW * CH) - 1) // (NW * CH)) * (NW * CH)
    x_flat = jnp.transpose(x, (0, 2, 1)).reshape(N_ * B_, C_)
    idx_pad = jnp.zeros((n_pad * K,), jnp.int32).at[: idx.shape[0]].set(idx)
    idx3 = idx_pad.reshape(NW, (n_pad // NW) // CH, CH * K)
    agg = _sc_gather_max(x_flat, idx3, n_pad)
    out = _tc_conv(agg, conv_w, conv_b.reshape(C_, 1), N_)
    return out.reshape(B_, conv_w.shape[0], N_)
